# Initial kernel scaffold; baseline (speedup 1.0000x reference)
import jax
import jax.numpy as jnp
from jax.experimental import pallas as pl


def kernel(pred, target, ignore_position):
    raise NotImplementedError("write your pallas kernel here")


# speedup vs baseline: 1.4464x; 1.4464x over previous
import functools

import jax
import jax.numpy as jnp
from jax import lax
from jax.experimental import pallas as pl
from jax.experimental.pallas import tpu as pltpu

K = 16
L = 32
P = 384 * 384
BS = 4
EXT = L + 2


def _ae_kernel(pred_ref, t_ref, ig_ref, out_ref, acc_ref, *, nb):
    b = pl.program_id(0)
    j = pl.program_id(1)

    @pl.when(j == 0)
    def _init():
        acc_ref[...] = jnp.zeros_like(acc_ref)

    @pl.when((b == 0) & (j == 0))
    def _init_out():
        out_ref[...] = jnp.zeros_like(out_ref)

    p = pred_ref[0]
    t = t_ref[0, 0]
    ig = ig_ref[0, 0]
    pb = p.shape[1]

    n2 = jnp.sum(p * p, axis=0, keepdims=True)
    inv = 1.0 / (jnp.sqrt(n2) + 1e-6)
    pn = p * inv
    psq = n2 * (inv * inv)
    ones = jnp.ones((1, pb), jnp.float32)
    ext = jnp.concatenate([pn, ones, psq], axis=0)

    ids = lax.broadcasted_iota(jnp.int32, (K, pb), 0) + 1
    mf = ((t == ids) & (ig == 0)).astype(jnp.float32)

    acc_ref[...] += lax.dot_general(
        mf, ext, (((1,), (1,)), ((), ())),
        preferred_element_type=jnp.float32)

    @pl.when(j == nb - 1)
    def _epilogue():
        acc = acc_ref[...]
        sums = acc[:, :L]
        cnt = acc[:, L:L + 1]
        sq = acc[:, L + 1:L + 2]
        present = cnt > 0.0
        pm = present.astype(jnp.float32)
        nf = jnp.sum(pm)
        cnt_safe = jnp.maximum(cnt, 1.0)
        ssum = jnp.sum(sums * sums, axis=1, keepdims=True)
        mse = (sq - ssum / cnt_safe) / (L * cnt_safe)
        pull_sum = jnp.sum(jnp.where(present, mse, 0.0))
        s_col = jnp.sum(sums, axis=1, keepdims=True) / cnt_safe
        onehot_cnt = (lax.broadcasted_iota(jnp.int32, (1, EXT), 1) == L
                      ).astype(jnp.float32)
        cnt_row = lax.dot_general(
            onehot_cnt, acc, (((1,), (1,)), ((), ())),
            preferred_element_type=jnp.float32)
        ones_row = jnp.ones((1, L), jnp.float32)
        s_row = lax.dot_general(
            ones_row, sums / cnt_safe, (((1,), (1,)), ((), ())),
            preferred_element_type=jnp.float32)
        pm_row = (cnt_row > 0.0).astype(jnp.float32)
        ds = s_row - s_col
        push_raw = jnp.sum(pm * pm_row * jnp.exp(-(ds * ds)))
        eps = 1e-6
        pull = jnp.where(nf > 0.0, pull_sum / (nf + eps), 0.0)
        push = jnp.where(nf > 1.0, push_raw / ((nf - 1.0) * nf + eps), 0.0)
        out_ref[...] += jnp.reshape(pull + 0.1 * push, (1, 1))


@jax.jit
def kernel(pred, target, ignore_position):
    nb = 16
    pb = P // nb
    predr = pred.reshape(BS, L, P)
    tr = target.reshape(BS, nb, 1, pb).astype(jnp.int32)
    igr = ignore_position.reshape(BS, nb, 1, pb).astype(jnp.int32)
    out = pl.pallas_call(
        functools.partial(_ae_kernel, nb=nb),
        grid=(BS, nb),
        in_specs=[
            pl.BlockSpec((1, L, pb), lambda b, j: (b, 0, j)),
            pl.BlockSpec((1, 1, 1, pb), lambda b, j: (b, j, 0, 0)),
            pl.BlockSpec((1, 1, 1, pb), lambda b, j: (b, j, 0, 0)),
        ],
        out_specs=pl.BlockSpec((1, 1), lambda b, j: (0, 0)),
        out_shape=jax.ShapeDtypeStruct((1, 1), jnp.float32),
        scratch_shapes=[pltpu.VMEM((K, EXT), jnp.float32)],
        compiler_params=pltpu.CompilerParams(
            dimension_semantics=("arbitrary", "arbitrary")),
    )(predr, tr, igr)
    return out[0, 0]


# speedup vs baseline: 1.6196x; 1.1197x over previous
import functools

import jax
import jax.numpy as jnp
from jax import lax
from jax.experimental import pallas as pl
from jax.experimental.pallas import tpu as pltpu

K = 16
L = 32
P = 384 * 384
BS = 4
EXT = L + 2


def _ae_kernel(pred_ref, t_ref, ig_ref, out_ref, acc_ref, *, nb):
    b = pl.program_id(0)
    j = pl.program_id(1)

    @pl.when(j == 0)
    def _init():
        acc_ref[...] = jnp.zeros_like(acc_ref)

    @pl.when((b == 0) & (j == 0))
    def _init_out():
        out_ref[...] = jnp.zeros_like(out_ref)

    p = pred_ref[0]
    t = t_ref[0, 0]
    ig = ig_ref[0, 0]
    pb = p.shape[1]

    n2 = jnp.sum(p * p, axis=0, keepdims=True)
    inv = 1.0 / (jnp.sqrt(n2) + 1e-6)
    pn = p * inv
    psq = n2 * (inv * inv)
    ones = jnp.ones((1, pb), jnp.float32)
    ext = jnp.concatenate([pn, ones, psq], axis=0)

    ids = lax.broadcasted_iota(jnp.int32, (K, pb), 0) + 1
    mf = ((t == ids) & (ig == 0)).astype(jnp.float32)

    acc_ref[...] += lax.dot_general(
        mf, ext, (((1,), (1,)), ((), ())),
        preferred_element_type=jnp.float32)

    @pl.when(j == nb - 1)
    def _epilogue():
        acc = acc_ref[...]
        sums = acc[:, :L]
        cnt = acc[:, L:L + 1]
        sq = acc[:, L + 1:L + 2]
        present = cnt > 0.0
        pm = present.astype(jnp.float32)
        nf = jnp.sum(pm)
        cnt_safe = jnp.maximum(cnt, 1.0)
        ssum = jnp.sum(sums * sums, axis=1, keepdims=True)
        mse = (sq - ssum / cnt_safe) / (L * cnt_safe)
        pull_sum = jnp.sum(jnp.where(present, mse, 0.0))
        s_col = jnp.sum(sums, axis=1, keepdims=True) / cnt_safe
        onehot_cnt = (lax.broadcasted_iota(jnp.int32, (1, EXT), 1) == L
                      ).astype(jnp.float32)
        cnt_row = lax.dot_general(
            onehot_cnt, acc, (((1,), (1,)), ((), ())),
            preferred_element_type=jnp.float32)
        ones_row = jnp.ones((1, L), jnp.float32)
        s_row = lax.dot_general(
            ones_row, sums / cnt_safe, (((1,), (1,)), ((), ())),
            preferred_element_type=jnp.float32)
        pm_row = (cnt_row > 0.0).astype(jnp.float32)
        ds = s_row - s_col
        push_raw = jnp.sum(pm * pm_row * jnp.exp(-(ds * ds)))
        eps = 1e-6
        pull = jnp.where(nf > 0.0, pull_sum / (nf + eps), 0.0)
        push = jnp.where(nf > 1.0, push_raw / ((nf - 1.0) * nf + eps), 0.0)
        out_ref[...] += jnp.reshape(pull + 0.1 * push, (1, 1))


@jax.jit
def kernel(pred, target, ignore_position):
    nb = 8
    pb = P // nb
    predr = pred.reshape(BS, L, P)
    tr = target.reshape(BS, nb, 1, pb).astype(jnp.int32)
    igr = ignore_position.reshape(BS, nb, 1, pb).astype(jnp.int32)
    out = pl.pallas_call(
        functools.partial(_ae_kernel, nb=nb),
        grid=(BS, nb),
        in_specs=[
            pl.BlockSpec((1, L, pb), lambda b, j: (b, 0, j)),
            pl.BlockSpec((1, 1, 1, pb), lambda b, j: (b, j, 0, 0)),
            pl.BlockSpec((1, 1, 1, pb), lambda b, j: (b, j, 0, 0)),
        ],
        out_specs=pl.BlockSpec((1, 1), lambda b, j: (0, 0)),
        out_shape=jax.ShapeDtypeStruct((1, 1), jnp.float32),
        scratch_shapes=[pltpu.VMEM((K, EXT), jnp.float32)],
        compiler_params=pltpu.CompilerParams(
            dimension_semantics=("arbitrary", "arbitrary")),
    )(predr, tr, igr)
    return out[0, 0]


# speedup vs baseline: 1.7117x; 1.0569x over previous
import functools

import jax
import jax.numpy as jnp
from jax import lax
from jax.experimental import pallas as pl
from jax.experimental.pallas import tpu as pltpu

K = 16
L = 32
P = 384 * 384
BS = 4
EXT = L + 2


def _ae_kernel(pred_ref, t_ref, ig_ref, out_ref, acc_ref, *, nb):
    b = pl.program_id(0)
    j = pl.program_id(1)

    @pl.when(j == 0)
    def _init():
        acc_ref[...] = jnp.zeros_like(acc_ref)

    @pl.when((b == 0) & (j == 0))
    def _init_out():
        out_ref[...] = jnp.zeros_like(out_ref)

    p = pred_ref[0]
    t = t_ref[0, 0]
    ig = ig_ref[0, 0]
    pb = p.shape[1]

    n2 = jnp.sum(p * p, axis=0, keepdims=True)
    inv = 1.0 / (jnp.sqrt(n2) + 1e-6)
    pn = p * inv
    psq = n2 * (inv * inv)
    ones = jnp.ones((1, pb), jnp.float32)
    ext = jnp.concatenate([pn, ones, psq], axis=0)

    ids = lax.broadcasted_iota(jnp.int32, (K, pb), 0) + 1
    mf = ((t == ids) & (ig == 0)).astype(jnp.float32)

    acc_ref[...] += lax.dot_general(
        mf, ext, (((1,), (1,)), ((), ())),
        preferred_element_type=jnp.float32)

    @pl.when(j == nb - 1)
    def _epilogue():
        acc = acc_ref[...]
        sums = acc[:, :L]
        cnt = acc[:, L:L + 1]
        sq = acc[:, L + 1:L + 2]
        present = cnt > 0.0
        pm = present.astype(jnp.float32)
        nf = jnp.sum(pm)
        cnt_safe = jnp.maximum(cnt, 1.0)
        ssum = jnp.sum(sums * sums, axis=1, keepdims=True)
        mse = (sq - ssum / cnt_safe) / (L * cnt_safe)
        pull_sum = jnp.sum(jnp.where(present, mse, 0.0))
        s_col = jnp.sum(sums, axis=1, keepdims=True) / cnt_safe
        onehot_cnt = (lax.broadcasted_iota(jnp.int32, (1, EXT), 1) == L
                      ).astype(jnp.float32)
        cnt_row = lax.dot_general(
            onehot_cnt, acc, (((1,), (1,)), ((), ())),
            preferred_element_type=jnp.float32)
        ones_row = jnp.ones((1, L), jnp.float32)
        s_row = lax.dot_general(
            ones_row, sums / cnt_safe, (((1,), (1,)), ((), ())),
            preferred_element_type=jnp.float32)
        pm_row = (cnt_row > 0.0).astype(jnp.float32)
        ds = s_row - s_col
        push_raw = jnp.sum(pm * pm_row * jnp.exp(-(ds * ds)))
        eps = 1e-6
        pull = jnp.where(nf > 0.0, pull_sum / (nf + eps), 0.0)
        push = jnp.where(nf > 1.0, push_raw / ((nf - 1.0) * nf + eps), 0.0)
        out_ref[...] += jnp.reshape(pull + 0.1 * push, (1, 1))


@jax.jit
def kernel(pred, target, ignore_position):
    nb = 4
    pb = P // nb
    predr = pred.reshape(BS, L, P)
    tr = target.reshape(BS, nb, 1, pb).astype(jnp.int32)
    igr = ignore_position.reshape(BS, nb, 1, pb).astype(jnp.int32)
    out = pl.pallas_call(
        functools.partial(_ae_kernel, nb=nb),
        grid=(BS, nb),
        in_specs=[
            pl.BlockSpec((1, L, pb), lambda b, j: (b, 0, j)),
            pl.BlockSpec((1, 1, 1, pb), lambda b, j: (b, j, 0, 0)),
            pl.BlockSpec((1, 1, 1, pb), lambda b, j: (b, j, 0, 0)),
        ],
        out_specs=pl.BlockSpec((1, 1), lambda b, j: (0, 0)),
        out_shape=jax.ShapeDtypeStruct((1, 1), jnp.float32),
        scratch_shapes=[pltpu.VMEM((K, EXT), jnp.float32)],
        compiler_params=pltpu.CompilerParams(
            dimension_semantics=("arbitrary", "arbitrary")),
    )(predr, tr, igr)
    return out[0, 0]


# speedup vs baseline: 1.7732x; 1.0359x over previous
import functools

import jax
import jax.numpy as jnp
from jax import lax
from jax.experimental import pallas as pl
from jax.experimental.pallas import tpu as pltpu

K = 16
L = 32
P = 384 * 384
BS = 4
EXT = L + 2


def _ae_kernel(pred_ref, t_ref, ig_ref, out_ref, acc_ref, *, nb):
    b = pl.program_id(0)
    j = pl.program_id(1)

    @pl.when(j == 0)
    def _init():
        acc_ref[...] = jnp.zeros_like(acc_ref)

    @pl.when((b == 0) & (j == 0))
    def _init_out():
        out_ref[...] = jnp.zeros_like(out_ref)

    p = pred_ref[0]
    t = t_ref[0, 0]
    ig = ig_ref[0, 0]
    pb = p.shape[1]

    n2 = jnp.sum(p * p, axis=0, keepdims=True)
    inv = 1.0 / (jnp.sqrt(n2) + 1e-6)
    pn = p * inv
    psq = n2 * (inv * inv)
    ones = jnp.ones((1, pb), jnp.float32)
    ext = jnp.concatenate([pn, ones, psq], axis=0)

    ids = lax.broadcasted_iota(jnp.int32, (K, pb), 0) + 1
    mf = ((t == ids) & (ig == 0)).astype(jnp.float32)

    acc_ref[...] += lax.dot_general(
        mf, ext, (((1,), (1,)), ((), ())),
        preferred_element_type=jnp.float32)

    @pl.when(j == nb - 1)
    def _epilogue():
        acc = acc_ref[...]
        sums = acc[:, :L]
        cnt = acc[:, L:L + 1]
        sq = acc[:, L + 1:L + 2]
        present = cnt > 0.0
        pm = present.astype(jnp.float32)
        nf = jnp.sum(pm)
        cnt_safe = jnp.maximum(cnt, 1.0)
        ssum = jnp.sum(sums * sums, axis=1, keepdims=True)
        mse = (sq - ssum / cnt_safe) / (L * cnt_safe)
        pull_sum = jnp.sum(jnp.where(present, mse, 0.0))
        s_col = jnp.sum(sums, axis=1, keepdims=True) / cnt_safe
        onehot_cnt = (lax.broadcasted_iota(jnp.int32, (1, EXT), 1) == L
                      ).astype(jnp.float32)
        cnt_row = lax.dot_general(
            onehot_cnt, acc, (((1,), (1,)), ((), ())),
            preferred_element_type=jnp.float32)
        ones_row = jnp.ones((1, L), jnp.float32)
        s_row = lax.dot_general(
            ones_row, sums / cnt_safe, (((1,), (1,)), ((), ())),
            preferred_element_type=jnp.float32)
        pm_row = (cnt_row > 0.0).astype(jnp.float32)
        ds = s_row - s_col
        push_raw = jnp.sum(pm * pm_row * jnp.exp(-(ds * ds)))
        eps = 1e-6
        pull = jnp.where(nf > 0.0, pull_sum / (nf + eps), 0.0)
        push = jnp.where(nf > 1.0, push_raw / ((nf - 1.0) * nf + eps), 0.0)
        out_ref[...] += jnp.reshape(pull + 0.1 * push, (1, 1))


@jax.jit
def kernel(pred, target, ignore_position):
    nb = 2
    pb = P // nb
    predr = pred.reshape(BS, L, P)
    tr = target.reshape(BS, nb, 1, pb).astype(jnp.int32)
    igr = ignore_position.reshape(BS, nb, 1, pb).astype(jnp.int32)
    out = pl.pallas_call(
        functools.partial(_ae_kernel, nb=nb),
        grid=(BS, nb),
        in_specs=[
            pl.BlockSpec((1, L, pb), lambda b, j: (b, 0, j)),
            pl.BlockSpec((1, 1, 1, pb), lambda b, j: (b, j, 0, 0)),
            pl.BlockSpec((1, 1, 1, pb), lambda b, j: (b, j, 0, 0)),
        ],
        out_specs=pl.BlockSpec((1, 1), lambda b, j: (0, 0)),
        out_shape=jax.ShapeDtypeStruct((1, 1), jnp.float32),
        scratch_shapes=[pltpu.VMEM((K, EXT), jnp.float32)],
        compiler_params=pltpu.CompilerParams(
            dimension_semantics=("arbitrary", "arbitrary")),
    )(predr, tr, igr)
    return out[0, 0]
